# Initial kernel scaffold; baseline (speedup 1.0000x reference)
import jax
import jax.numpy as jnp
from jax.experimental import pallas as pl


def kernel(x, epoch, sample, phi_ikp_inner, dphi_ikp_inner, ddphi_ikp_inner, cached_x):
    raise NotImplementedError("write your pallas kernel here")


# speedup vs baseline: 18.5997x; 18.5997x over previous
import jax
import jax.numpy as jnp
from jax import lax
from jax.experimental import pallas as pl
from jax.experimental.pallas import tpu as pltpu
from jax.experimental.pallas import tpu_sc as plsc

_N_WIDTH = 512
_N_ORDER = 2
_N_ELEMENTS = 128
_N_NODES = _N_ELEMENTS * _N_ORDER + 1
_X_MIN = -1.0
_X_MAX = 1.0

_NC = 2
_NS = 16
_NW = _NC * _NS
_ROWS_PER_W = _N_WIDTH // _NW
_L = 16


def _phi_body(x_hbm, row_hbm, out_hbm, x_v, chunk_v):
    c = lax.axis_index("c")
    s = lax.axis_index("s")
    wid = s * _NC + c
    base = wid * _ROWS_PER_W

    pltpu.sync_copy(x_hbm, x_v.at[pl.ds(0, 1)])
    pltpu.sync_copy(row_hbm.at[pl.ds(base, _ROWS_PER_W)], chunk_v)

    xv = jnp.full((_L,), x_v[...][0], jnp.float32)
    x_shift = (_N_NODES - 1) * (xv - _X_MIN) / (_X_MAX - _X_MIN)
    iq = (x_shift / _N_ORDER).astype(jnp.int32)
    iq = jnp.maximum(jnp.minimum(iq, _N_ELEMENTS - 1), 0)
    c0 = iq * _N_ORDER
    xt = x_shift - (c0.astype(jnp.float32) + 1.0)

    p0 = (xt / -1.0) * ((xt - 1.0) / -2.0)
    p1 = (xt + 1.0) * ((xt - 1.0) / -1.0)
    p2 = ((xt + 1.0) / 2.0) * xt

    rows = lax.iota(jnp.int32, _L)
    plsc.store_scatter(chunk_v, [rows, c0], p0)
    plsc.store_scatter(chunk_v, [rows, c0 + 1], p1)
    plsc.store_scatter(chunk_v, [rows, c0 + 2], p2)

    pltpu.sync_copy(chunk_v, out_hbm.at[pl.ds(base, _ROWS_PER_W)])


_phi_sc = pl.kernel(
    _phi_body,
    mesh=plsc.VectorSubcoreMesh(core_axis_name="c", subcore_axis_name="s"),
    out_type=jax.ShapeDtypeStruct((_N_WIDTH, _N_NODES), jnp.float32),
    scratch_types=[
        pltpu.VMEM((_L,), jnp.float32),
        pltpu.VMEM((_ROWS_PER_W, _N_NODES), jnp.float32),
    ],
    compiler_params=pltpu.CompilerParams(needs_layout_passes=False),
)


def kernel(x, epoch, sample, phi_ikp_inner, dphi_ikp_inner, ddphi_ikp_inner,
           cached_x):
    s0 = jnp.asarray(sample, jnp.int32)
    row = lax.dynamic_slice(
        phi_ikp_inner, (s0, jnp.int32(0), jnp.int32(0)),
        (1, _N_WIDTH, _N_NODES))[0]
    out = _phi_sc(x.astype(jnp.float32), row)
    return out[None]
